# baseline (speedup 1.0000x reference)
import functools

import jax
import jax.numpy as jnp
from jax import lax
from jax.experimental import pallas as pl
from jax.experimental.pallas import tpu as pltpu
from jax.experimental.pallas import tpu_sc as plsc

_VOCAB = 1000000
_D = 64
_B = 4096
_NSENT = 5
_LSEQ = 50

_NC, _NSUB = 2, 16
_NW = _NC * _NSUB
_SEGS = _B * _NSENT
_ROWS = _SEGS * _LSEQ
_SEGS_PER_W = _SEGS // _NW
_ROWS_PER_W = _ROWS // _NW
_CHUNK_SEGS = 8
_CHUNK_ROWS = _CHUNK_SEGS * _LSEQ
_N_CHUNKS = _SEGS_PER_W // _CHUNK_SEGS


_BLK = 128
_NFULL = _VOCAB // _BLK
_TAIL = _VOCAB - _NFULL * _BLK
_TAIL_WORKER = _NFULL % _NW


def _sc_detile(table):
    tT = table.T
    tail_pairs = table[_NFULL * _BLK :, :].reshape(_TAIL // 2, 2 * _D)
    mesh = plsc.VectorSubcoreMesh(core_axis_name="c", subcore_axis_name="s")

    @functools.partial(
        pl.kernel,
        out_type=jax.ShapeDtypeStruct((_VOCAB // 2, 2 * _D), jnp.float32),
        mesh=mesh,
        scratch_types=[
            pltpu.VMEM((_D, _BLK), jnp.float32),
            pltpu.VMEM((_D, _BLK), jnp.float32),
            pltpu.VMEM((_BLK // 2, 2 * _D), jnp.float32),
            pltpu.VMEM((_BLK // 2, 2 * _D), jnp.float32),
            pltpu.SemaphoreType.DMA,
            pltpu.SemaphoreType.DMA,
            pltpu.SemaphoreType.DMA,
            pltpu.SemaphoreType.DMA,
        ],
        compiler_params=pltpu.CompilerParams(
            use_tc_tiling_on_sc=True, needs_layout_passes=False
        ),
    )
    def detile_kernel(tT_hbm, tailp_hbm, out_hbm, in0, in1, o0, o1,
                      si0, si1, so0, so1):
        wid = lax.axis_index("s") * _NC + lax.axis_index("c")
        nblk = (_NFULL - wid + _NW - 1) // _NW

        def blk_of(k):
            return wid + k * _NW

        def start_in(k, buf, sem):
            pltpu.async_copy(
                tT_hbm.at[:, pl.ds(blk_of(k) * _BLK, _BLK)], buf, sem
            )

        def wait_in(buf, sem):
            pltpu.make_async_copy(
                tT_hbm.at[:, pl.ds(0, _BLK)], buf, sem
            ).wait()

        def start_out(k, buf, sem):
            pltpu.async_copy(
                buf, out_hbm.at[pl.ds(blk_of(k) * (_BLK // 2), _BLK // 2)], sem
            )

        def wait_out(buf, sem):
            pltpu.make_async_copy(
                buf, out_hbm.at[pl.ds(0, _BLK // 2)], sem
            ).wait()

        def transpose(in_ref, o_ref, npairs):
            def p_body(p, carry):
                r0 = 2 * p
                for g in range(4):
                    jv = jnp.arange(16, dtype=jnp.int32) + 16 * g
                    o_ref[p, pl.ds(16 * g, 16)] = plsc.load_gather(
                        in_ref, [jv, jnp.full((16,), r0, jnp.int32)]
                    )
                    o_ref[p, pl.ds(_D + 16 * g, 16)] = plsc.load_gather(
                        in_ref, [jv, jnp.full((16,), r0 + 1, jnp.int32)]
                    )
                return carry

            lax.fori_loop(0, npairs, p_body, 0, unroll=2)

        start_in(0, in0, si0)

        def pair_body(k2, carry):
            k0 = 2 * k2
            k1 = k0 + 1

            @pl.when(k1 < nblk)
            def _():
                start_in(k1, in1, si1)

            wait_in(in0, si0)

            @pl.when(k2 > 0)
            def _():
                wait_out(o0, so0)

            transpose(in0, o0, _BLK // 2)
            start_out(k0, o0, so0)

            @pl.when(k0 + 2 < nblk)
            def _():
                start_in(k0 + 2, in0, si0)

            @pl.when(k1 < nblk)
            def _():
                wait_in(in1, si1)

                @pl.when(k2 > 0)
                def _():
                    wait_out(o1, so1)

                transpose(in1, o1, _BLK // 2)
                start_out(k1, o1, so1)

            return carry

        lax.fori_loop(0, (nblk + 1) // 2, pair_body, 0)
        wait_out(o0, so0)
        wait_out(o1, so1)

        @pl.when(wid == _TAIL_WORKER)
        def _():
            pltpu.sync_copy(tailp_hbm, o0.at[pl.ds(0, _TAIL // 2)])
            pltpu.sync_copy(
                o0.at[pl.ds(0, _TAIL // 2)],
                out_hbm.at[pl.ds(_NFULL * (_BLK // 2), _TAIL // 2)],
            )

    return detile_kernel(tT, tail_pairs)


def _sc_segment_sums(table, flat_idx):
    mesh = plsc.VectorSubcoreMesh(core_axis_name="c", subcore_axis_name="s")

    @functools.partial(
        pl.kernel,
        out_type=jax.ShapeDtypeStruct((_SEGS, _D), jnp.float32),
        mesh=mesh,
        scratch_types=[
            pltpu.VMEM((_ROWS_PER_W,), jnp.int32),
            pltpu.VMEM((_CHUNK_ROWS, _D), jnp.float32),
            pltpu.VMEM((_CHUNK_ROWS, _D), jnp.float32),
            pltpu.VMEM((_SEGS_PER_W, _D), jnp.float32),
            pltpu.SemaphoreType.DMA,
            pltpu.SemaphoreType.DMA,
        ],
        compiler_params=pltpu.CompilerParams(use_tc_tiling_on_sc=False),
    )
    def sums_kernel(table_hbm, idx_hbm, out_hbm, idx_v, rows0, rows1, acc_v,
                    sem0, sem1):
        wid = lax.axis_index("s") * _NC + lax.axis_index("c")
        pltpu.sync_copy(idx_hbm.at[pl.ds(wid * _ROWS_PER_W, _ROWS_PER_W)], idx_v)

        def start_gather(c, rows, sem):
            return pltpu.async_copy(
                table_hbm.at[idx_v.at[pl.ds(c * _CHUNK_ROWS, _CHUNK_ROWS)]],
                rows, sem,
            )

        def wait_slot0():
            pltpu.make_async_copy(
                table_hbm.at[pl.ds(0, _CHUNK_ROWS)], rows0, sem0
            ).wait()

        def accumulate(rows, c):
            def seg_body(s, carry2):
                def row_body(r, accs):
                    b = s * _LSEQ + r
                    return tuple(
                        accs[k] + rows[b, pl.ds(16 * k, 16)] for k in range(4)
                    )

                z = jnp.zeros((16,), jnp.float32)
                a = lax.fori_loop(0, _LSEQ, row_body, (z, z, z, z), unroll=10)
                for k in range(4):
                    acc_v[c * _CHUNK_SEGS + s, pl.ds(16 * k, 16)] = a[k]
                return carry2

            lax.fori_loop(0, _CHUNK_SEGS, seg_body, 0)

        start_gather(0, rows0, sem0)

        def pair_body(c2, carry):
            c0 = 2 * c2
            d1 = start_gather(c0 + 1, rows1, sem1)
            wait_slot0()
            accumulate(rows0, c0)

            @pl.when(c2 < _N_CHUNKS // 2 - 1)
            def _():
                start_gather(c0 + 2, rows0, sem0)

            d1.wait()
            accumulate(rows1, c0 + 1)
            return carry

        lax.fori_loop(0, _N_CHUNKS // 2, pair_body, 0)
        pltpu.sync_copy(acc_v, out_hbm.at[pl.ds(wid * _SEGS_PER_W, _SEGS_PER_W)])

    return sums_kernel(table, flat_idx)


def _tc_finish(sums):
    def body(x_ref, cos_ref, pred_ref):
        x = x_ref[...] * (1.0 / _LSEQ)
        anchor = x[0:_B]
        a2 = jnp.sum(anchor * anchor, axis=-1, keepdims=True)
        nums = []
        o2s = []
        for k in range(1, _NSENT):
            o = x[k * _B : (k + 1) * _B]
            nums.append(jnp.sum(anchor * o, axis=-1, keepdims=True))
            o2s.append(jnp.sum(o * o, axis=-1, keepdims=True))
        num = jnp.concatenate(nums, axis=1)
        on = jnp.sqrt(jnp.concatenate(o2s, axis=1))
        an = jnp.sqrt(a2)
        cos = num / (an * on + 1e-8)
        cos_ref[...] = cos
        m = jnp.max(cos, axis=1, keepdims=True)
        e = jnp.exp(cos - m)
        pred_ref[...] = e / jnp.sum(e, axis=1, keepdims=True)

    return pl.pallas_call(
        body,
        out_shape=(
            jax.ShapeDtypeStruct((_B, _NSENT - 1), jnp.float32),
            jax.ShapeDtypeStruct((_B, _NSENT - 1), jnp.float32),
        ),
    )(sums)


def kernel(inputs, table):
    flat_idx = inputs.transpose(1, 0, 2).reshape(-1)
    table_lin = _sc_detile(table).reshape(_VOCAB, _D)
    sums = _sc_segment_sums(table_lin, flat_idx)
    return _tc_finish(sums)


# speedup vs baseline: 4.6200x; 4.6200x over previous
import functools

import jax
import jax.numpy as jnp
from jax import lax
from jax.experimental import pallas as pl
from jax.experimental.pallas import tpu as pltpu
from jax.experimental.pallas import tpu_sc as plsc

_VOCAB = 1000000
_D = 64
_B = 4096
_NSENT = 5
_LSEQ = 50

_NC, _NSUB = 2, 16
_NW = _NC * _NSUB
_SEGS = _B * _NSENT
_ROWS = _SEGS * _LSEQ
_SEGS_PER_W = _SEGS // _NW
_ROWS_PER_W = _ROWS // _NW
_CHUNK_SEGS = 8
_CHUNK_ROWS = _CHUNK_SEGS * _LSEQ
_N_CHUNKS = _SEGS_PER_W // _CHUNK_SEGS


_BLK = 128
_NFULL = _VOCAB // _BLK
_TAIL = _VOCAB - _NFULL * _BLK
_TAIL_WORKER = _NFULL % _NW


def _sc_detile(table):
    tT = table.T
    tail_pairs = table[_NFULL * _BLK :, :].reshape(_TAIL // 2, 2 * _D)
    mesh = plsc.VectorSubcoreMesh(core_axis_name="c", subcore_axis_name="s")

    @functools.partial(
        pl.kernel,
        out_type=jax.ShapeDtypeStruct((_VOCAB // 2, 2 * _D), jnp.float32),
        mesh=mesh,
        scratch_types=[
            pltpu.VMEM((_D, _BLK), jnp.float32),
            pltpu.VMEM((_D, _BLK), jnp.float32),
            pltpu.VMEM((_BLK // 2, 2 * _D), jnp.float32),
            pltpu.VMEM((_BLK // 2, 2 * _D), jnp.float32),
            pltpu.SemaphoreType.DMA,
            pltpu.SemaphoreType.DMA,
            pltpu.SemaphoreType.DMA,
            pltpu.SemaphoreType.DMA,
        ],
        compiler_params=pltpu.CompilerParams(
            use_tc_tiling_on_sc=True, needs_layout_passes=False
        ),
    )
    def detile_kernel(tT_hbm, tailp_hbm, out_hbm, in0, in1, o0, o1,
                      si0, si1, so0, so1):
        wid = lax.axis_index("s") * _NC + lax.axis_index("c")
        nblk = (_NFULL - wid + _NW - 1) // _NW

        def blk_of(k):
            return wid + k * _NW

        def start_in(k, buf, sem):
            pltpu.async_copy(
                tT_hbm.at[:, pl.ds(blk_of(k) * _BLK, _BLK)], buf, sem
            )

        def wait_in(buf, sem):
            pltpu.make_async_copy(
                tT_hbm.at[:, pl.ds(0, _BLK)], buf, sem
            ).wait()

        def start_out(k, buf, sem):
            pltpu.async_copy(
                buf, out_hbm.at[pl.ds(blk_of(k) * (_BLK // 2), _BLK // 2)], sem
            )

        def wait_out(buf, sem):
            pltpu.make_async_copy(
                buf, out_hbm.at[pl.ds(0, _BLK // 2)], sem
            ).wait()

        def transpose(in_ref, o_ref, npairs):
            @functools.partial(plsc.parallel_loop, 0, npairs, unroll=4)
            def _(p):
                r0 = 2 * p
                for g in range(4):
                    jv = jnp.arange(16, dtype=jnp.int32) + 16 * g
                    o_ref[p, pl.ds(16 * g, 16)] = plsc.load_gather(
                        in_ref, [jv, jnp.full((16,), r0, jnp.int32)]
                    )
                    o_ref[p, pl.ds(_D + 16 * g, 16)] = plsc.load_gather(
                        in_ref, [jv, jnp.full((16,), r0 + 1, jnp.int32)]
                    )

        start_in(0, in0, si0)

        def pair_body(k2, carry):
            k0 = 2 * k2
            k1 = k0 + 1

            @pl.when(k1 < nblk)
            def _():
                start_in(k1, in1, si1)

            wait_in(in0, si0)

            @pl.when(k2 > 0)
            def _():
                wait_out(o0, so0)

            transpose(in0, o0, _BLK // 2)
            start_out(k0, o0, so0)

            @pl.when(k0 + 2 < nblk)
            def _():
                start_in(k0 + 2, in0, si0)

            @pl.when(k1 < nblk)
            def _():
                wait_in(in1, si1)

                @pl.when(k2 > 0)
                def _():
                    wait_out(o1, so1)

                transpose(in1, o1, _BLK // 2)
                start_out(k1, o1, so1)

            return carry

        lax.fori_loop(0, (nblk + 1) // 2, pair_body, 0)
        wait_out(o0, so0)
        wait_out(o1, so1)

        @pl.when(wid == _TAIL_WORKER)
        def _():
            pltpu.sync_copy(tailp_hbm, o0.at[pl.ds(0, _TAIL // 2)])
            pltpu.sync_copy(
                o0.at[pl.ds(0, _TAIL // 2)],
                out_hbm.at[pl.ds(_NFULL * (_BLK // 2), _TAIL // 2)],
            )

    return detile_kernel(tT, tail_pairs)


def _sc_segment_sums(table, flat_idx):
    mesh = plsc.VectorSubcoreMesh(core_axis_name="c", subcore_axis_name="s")

    @functools.partial(
        pl.kernel,
        out_type=jax.ShapeDtypeStruct((_SEGS, _D), jnp.float32),
        mesh=mesh,
        scratch_types=[
            pltpu.VMEM((_ROWS_PER_W,), jnp.int32),
            pltpu.VMEM((_CHUNK_ROWS, _D), jnp.float32),
            pltpu.VMEM((_CHUNK_ROWS, _D), jnp.float32),
            pltpu.VMEM((_SEGS_PER_W, _D), jnp.float32),
            pltpu.SemaphoreType.DMA,
            pltpu.SemaphoreType.DMA,
        ],
        compiler_params=pltpu.CompilerParams(use_tc_tiling_on_sc=False),
    )
    def sums_kernel(table_hbm, idx_hbm, out_hbm, idx_v, rows0, rows1, acc_v,
                    sem0, sem1):
        wid = lax.axis_index("s") * _NC + lax.axis_index("c")
        pltpu.sync_copy(idx_hbm.at[pl.ds(wid * _ROWS_PER_W, _ROWS_PER_W)], idx_v)

        def start_gather(c, rows, sem):
            return pltpu.async_copy(
                table_hbm.at[idx_v.at[pl.ds(c * _CHUNK_ROWS, _CHUNK_ROWS)]],
                rows, sem,
            )

        def wait_slot0():
            pltpu.make_async_copy(
                table_hbm.at[pl.ds(0, _CHUNK_ROWS)], rows0, sem0
            ).wait()

        def accumulate(rows, c):
            def seg_body(s, carry2):
                def row_body(r, accs):
                    b = s * _LSEQ + r
                    return tuple(
                        accs[k] + rows[b, pl.ds(16 * k, 16)] for k in range(4)
                    )

                z = jnp.zeros((16,), jnp.float32)
                a = lax.fori_loop(0, _LSEQ, row_body, (z, z, z, z), unroll=10)
                for k in range(4):
                    acc_v[c * _CHUNK_SEGS + s, pl.ds(16 * k, 16)] = a[k]
                return carry2

            lax.fori_loop(0, _CHUNK_SEGS, seg_body, 0)

        start_gather(0, rows0, sem0)

        def pair_body(c2, carry):
            c0 = 2 * c2
            d1 = start_gather(c0 + 1, rows1, sem1)
            wait_slot0()
            accumulate(rows0, c0)

            @pl.when(c2 < _N_CHUNKS // 2 - 1)
            def _():
                start_gather(c0 + 2, rows0, sem0)

            d1.wait()
            accumulate(rows1, c0 + 1)
            return carry

        lax.fori_loop(0, _N_CHUNKS // 2, pair_body, 0)
        pltpu.sync_copy(acc_v, out_hbm.at[pl.ds(wid * _SEGS_PER_W, _SEGS_PER_W)])

    return sums_kernel(table, flat_idx)


def _tc_finish(sums):
    def body(x_ref, cos_ref, pred_ref):
        x = x_ref[...] * (1.0 / _LSEQ)
        anchor = x[0:_B]
        a2 = jnp.sum(anchor * anchor, axis=-1, keepdims=True)
        nums = []
        o2s = []
        for k in range(1, _NSENT):
            o = x[k * _B : (k + 1) * _B]
            nums.append(jnp.sum(anchor * o, axis=-1, keepdims=True))
            o2s.append(jnp.sum(o * o, axis=-1, keepdims=True))
        num = jnp.concatenate(nums, axis=1)
        on = jnp.sqrt(jnp.concatenate(o2s, axis=1))
        an = jnp.sqrt(a2)
        cos = num / (an * on + 1e-8)
        cos_ref[...] = cos
        m = jnp.max(cos, axis=1, keepdims=True)
        e = jnp.exp(cos - m)
        pred_ref[...] = e / jnp.sum(e, axis=1, keepdims=True)

    return pl.pallas_call(
        body,
        out_shape=(
            jax.ShapeDtypeStruct((_B, _NSENT - 1), jnp.float32),
            jax.ShapeDtypeStruct((_B, _NSENT - 1), jnp.float32),
        ),
    )(sums)


def kernel(inputs, table):
    flat_idx = inputs.transpose(1, 0, 2).reshape(-1)
    table_lin = _sc_detile(table).reshape(_VOCAB, _D)
    sums = _sc_segment_sums(table_lin, flat_idx)
    return _tc_finish(sums)
